# baseline (speedup 1.0000x reference)
import functools

import jax
import jax.numpy as jnp
from jax import lax
from jax.experimental import pallas as pl
from jax.experimental.pallas import tpu as pltpu
from jax.experimental.pallas import tpu_sc as plsc

_ROWS = 16384
_FEATS = 128
_CHUNK = 128
_NCHUNKS = _ROWS // (2 * 16) // _CHUNK
_UNROLL = 8


def _sc_body(
    x_hbm, aidx_hbm, a_hbm, o_hbm, xbuf0, xbuf1, obuf0, obuf1, aidx_v, a_v, insem, outsem
):
    nc = 2
    ns = 16
    wid = lax.axis_index("s") * nc + lax.axis_index("c")
    rows_per_w = _ROWS // (nc * ns)
    base = wid * rows_per_w

    pltpu.sync_copy(aidx_hbm, aidx_v)
    pltpu.sync_copy(a_hbm, a_v)

    xbufs = (xbuf0, xbuf1)
    obufs = (obuf0, obuf1)

    def get_in(kc):
        return pltpu.make_async_copy(
            x_hbm.at[pl.ds(base + kc * _CHUNK, _CHUNK)],
            xbufs[kc % 2],
            insem.at[kc % 2],
        )

    def put_out(kc):
        return pltpu.make_async_copy(
            obufs[kc % 2],
            o_hbm.at[pl.ds(base + kc * _CHUNK, _CHUNK)],
            outsem.at[kc % 2],
        )

    get_in(0).start()
    for kc in range(_NCHUNKS):
        if kc + 1 < _NCHUNKS:
            get_in(kc + 1).start()
        get_in(kc).wait()
        if kc >= 2:
            put_out(kc - 2).wait()
        xbuf = xbufs[kc % 2]
        obuf = obufs[kc % 2]

        for j in range(_FEATS // 16):
            idx0 = aidx_v[pl.ds(j * 16, 16)]
            a_j = a_v[pl.ds(j * 16, 16)]

            @plsc.parallel_loop(0, _CHUNK, 1, unroll=_UNROLL)
            def row_body(r, xbuf=xbuf, obuf=obuf, idx0=idx0, a_j=a_j, j=j):
                ridx = jnp.full((16,), r, dtype=jnp.int32)
                g = plsc.load_gather(xbuf, [ridx, idx0])
                z = g - a_j
                y = 1.0 / (1.0 + jnp.exp(-z))
                obuf[r, pl.ds(j * 16, 16)] = y

        put_out(kc).start()
    if _NCHUNKS >= 2:
        put_out(_NCHUNKS - 2).wait()
    put_out(_NCHUNKS - 1).wait()


@jax.jit
def kernel(x, a, a_index):
    n, d = x.shape
    mesh = plsc.VectorSubcoreMesh(core_axis_name="c", subcore_axis_name="s")
    k = functools.partial(
        pl.kernel,
        mesh=mesh,
        compiler_params=pltpu.CompilerParams(needs_layout_passes=False),
        out_type=jax.ShapeDtypeStruct((n, d), x.dtype),
        scratch_types=[
            pltpu.VMEM((_CHUNK, d), x.dtype),
            pltpu.VMEM((_CHUNK, d), x.dtype),
            pltpu.VMEM((_CHUNK, d), x.dtype),
            pltpu.VMEM((_CHUNK, d), x.dtype),
            pltpu.VMEM((d,), jnp.int32),
            pltpu.VMEM((d,), x.dtype),
            pltpu.SemaphoreType.DMA((2,)),
            pltpu.SemaphoreType.DMA((2,)),
        ],
    )(_sc_body)
    return k(x, a_index, a.reshape(d))


# speedup vs baseline: 1.0177x; 1.0177x over previous
import functools

import jax
import jax.numpy as jnp
from jax import lax
from jax.experimental import pallas as pl
from jax.experimental.pallas import tpu as pltpu
from jax.experimental.pallas import tpu_sc as plsc

_ROWS = 16384
_FEATS = 128
_CHUNK = 128
_NCHUNKS = _ROWS // (2 * 16) // _CHUNK
_UNROLL = 8


def _sc_body(
    x_hbm, aidx_hbm, a_hbm, o_hbm,
    xbuf0, xbuf1, obuf0, obuf1, aidx_v, a_v, insem, outsem,
):
    nc = 2
    ns = 16
    wid = lax.axis_index("s") * nc + lax.axis_index("c")
    rows_per_w = _ROWS // (nc * ns)
    base = wid * rows_per_w
    cw = _CHUNK * _FEATS

    pltpu.sync_copy(aidx_hbm, aidx_v)
    pltpu.sync_copy(a_hbm, a_v)

    xbufs = (xbuf0, xbuf1)
    obufs = (obuf0, obuf1)

    def get_in(kc):
        return pltpu.make_async_copy(
            x_hbm.at[pl.ds((base + kc * _CHUNK) * _FEATS, cw)],
            xbufs[kc % 2],
            insem.at[kc % 2],
        )

    def put_out(kc):
        return pltpu.make_async_copy(
            obufs[kc % 2],
            o_hbm.at[pl.ds((base + kc * _CHUNK) * _FEATS, cw)],
            outsem.at[kc % 2],
        )

    get_in(0).start()
    for kc in range(_NCHUNKS):
        if kc + 1 < _NCHUNKS:
            get_in(kc + 1).start()
        get_in(kc).wait()
        if kc >= 2:
            put_out(kc - 2).wait()
        xbuf = xbufs[kc % 2]
        obuf = obufs[kc % 2]

        for j in range(_FEATS // 16):
            idx0 = aidx_v[pl.ds(j * 16, 16)]
            a_j = a_v[pl.ds(j * 16, 16)]

            @plsc.parallel_loop(0, _CHUNK, 1, unroll=_UNROLL)
            def row_body(r, xbuf=xbuf, obuf=obuf, idx0=idx0, a_j=a_j, j=j):
                g = plsc.load_gather(xbuf, [idx0 + r * _FEATS])
                z = g - a_j
                y = 1.0 / (1.0 + jnp.exp(-z))
                obuf[pl.ds(r * _FEATS + j * 16, 16)] = y

        put_out(kc).start()
    if _NCHUNKS >= 2:
        put_out(_NCHUNKS - 2).wait()
    put_out(_NCHUNKS - 1).wait()


@jax.jit
def kernel(x, a, a_index):
    n, d = x.shape
    mesh = plsc.VectorSubcoreMesh(core_axis_name="c", subcore_axis_name="s")
    k = functools.partial(
        pl.kernel,
        mesh=mesh,
        compiler_params=pltpu.CompilerParams(needs_layout_passes=False),
        out_type=jax.ShapeDtypeStruct((n * d,), x.dtype),
        scratch_types=[
            pltpu.VMEM((_CHUNK * d,), x.dtype),
            pltpu.VMEM((_CHUNK * d,), x.dtype),
            pltpu.VMEM((_CHUNK * d,), x.dtype),
            pltpu.VMEM((_CHUNK * d,), x.dtype),
            pltpu.VMEM((d,), jnp.int32),
            pltpu.VMEM((d,), x.dtype),
            pltpu.SemaphoreType.DMA((2,)),
            pltpu.SemaphoreType.DMA((2,)),
        ],
    )(_sc_body)
    return k(x.reshape(n * d), a_index, a.reshape(d)).reshape(n, d)
